# baseline (speedup 1.0000x reference)
import functools

import jax
import jax.numpy as jnp
from jax import lax
from jax.experimental import pallas as pl
from jax.experimental.pallas import tpu as pltpu
from jax.experimental.pallas import tpu_sc as plsc

NN = 100000
NP = 102400
EE = 6400000
EP = 6553600
NW = 32
EPW = EP // NW
BK = 2048
NB = EPW // BK
CH = 128
NCH = BK // CH
NPW = NP // NW
NLVL = 8


def _take16(x, idx):
    return lax.gather(
        x, idx[:, None],
        dimension_numbers=lax.GatherDimensionNumbers(
            offset_dims=(), collapsed_slice_dims=(0,), start_index_map=(0,)),
        slice_sizes=(1,),
        mode=lax.GatherScatterMode.PROMISE_IN_BOUNDS)


NB0 = 100
NB1 = 100


def _scatter_body(h_hbm, src_hbm, dst_hbm, neg_hbm, out_hbm,
                  agg, srcb0, dstb0, msgb0, srcb1, dstb1, msgb1, hsh, sem):
    cid = lax.axis_index("c")
    sid = lax.axis_index("s")
    wid = sid * 2 + cid
    base_blk = sid * (NB0 + NB1) + cid * NB0
    nblk = NB0 + cid * (NB1 - NB0)
    base = base_blk * BK

    @pl.when(sid == 0)
    def _():
        pltpu.sync_copy(h_hbm, hsh)

    pltpu.sync_copy(neg_hbm, agg)
    plsc.subcore_barrier()
    iota = lax.iota(jnp.int32, 16)
    bufs = ((srcb0, dstb0, msgb0), (srcb1, dstb1, msgb1))

    def _stage(b, p):
        sb, db, mb = bufs[p]
        off = base + b * BK
        pltpu.sync_copy(src_hbm.at[pl.ds(off, BK)], sb)
        pltpu.sync_copy(dst_hbm.at[pl.ds(off, BK)], db)
        for c in range(NCH):
            pltpu.async_copy(hsh.at[sb.at[pl.ds(c * CH, CH)]],
                             mb.at[pl.ds(c * CH, CH)], sem)

    def _wait(p):
        sb, db, mb = bufs[p]
        for c in range(NCH):
            pltpu.make_async_copy(hsh.at[sb.at[pl.ds(c * CH, CH)]],
                                  mb.at[pl.ds(c * CH, CH)], sem).wait()

    def _compute(p):
        db, mb = bufs[p][1], bufs[p][2]

        def vec(j, _):
            d = db[pl.ds(j * 16, 16)]
            m = mb[pl.ds(j * 16, 16)] + 1.0
            old = plsc.load_gather(agg, [d])
            plsc.store_scatter(agg, [d], jnp.maximum(old, m))
            chk = plsc.load_gather(agg, [d])

            @pl.when(jnp.any(chk < m))
            def _():
                k, v = plsc.sort_key_val(d, m)
                for s in (1, 2, 4, 8):
                    idx = jnp.maximum(iota - s, 0)
                    ks = _take16(k, idx)
                    vs = _take16(v, idx)
                    v = jnp.where((iota >= s) & (ks == k),
                                  jnp.maximum(v, vs), v)
                kl = _take16(k, jnp.minimum(iota + 1, 15))
                last = (k != kl) | (iota == 15)
                prev = plsc.load_gather(agg, [k])
                plsc.store_scatter(agg, [k], jnp.maximum(prev, v), mask=last)

            return 0

        lax.fori_loop(0, BK // 16, vec, 0)

    _stage(0, 0)

    def pair(t, _):
        for phase in range(2):
            b = t * 2 + phase
            _wait(phase)

            @pl.when(b + 1 < nblk)
            def _():
                _stage(b + 1, 1 - phase)

            _compute(phase)
        return 0

    lax.fori_loop(0, nblk // 2, pair, 0)
    pltpu.sync_copy(agg, out_hbm.at[wid])


def _apply_body(h_hbm, aggs_hbm, lvl_hbm, ivec_hbm, out_hbm,
                hbuf, lbuf, acc, tmp, ivec, sem):
    wid = lax.axis_index("s") * 2 + lax.axis_index("c")
    base = wid * NPW
    pltpu.sync_copy(h_hbm.at[pl.ds(base, NPW)], hbuf)
    pltpu.sync_copy(lvl_hbm.at[pl.ds(base, NPW)], lbuf)
    pltpu.sync_copy(ivec_hbm, ivec)
    pltpu.sync_copy(aggs_hbm.at[0, pl.ds(base, NPW)], acc)
    for s in range(1, NW):
        pltpu.sync_copy(aggs_hbm.at[s, pl.ds(base, NPW)], tmp)

        def mx(j, _):
            sl = pl.ds(j * 16, 16)
            acc[sl] = jnp.maximum(acc[sl], tmp[sl])
            return 0

        lax.fori_loop(0, NPW // 16, mx, 0)
    iv = ivec[...]

    def sel(j, _):
        sl = pl.ds(j * 16, 16)
        hbuf[sl] = jnp.where(lbuf[sl] == iv, acc[sl], hbuf[sl])
        return 0

    lax.fori_loop(0, NPW // 16, sel, 0)
    pltpu.sync_copy(hbuf, out_hbm.at[pl.ds(base, NPW)])


_MESH = plsc.VectorSubcoreMesh(core_axis_name="c", subcore_axis_name="s")
_CPARAMS = pltpu.CompilerParams(needs_layout_passes=False)

_scatter = functools.partial(
    pl.kernel,
    out_type=jax.ShapeDtypeStruct((NW, NP), jnp.float32),
    mesh=_MESH,
    compiler_params=_CPARAMS,
    scratch_types=[
        pltpu.VMEM((NP,), jnp.float32),
        pltpu.VMEM((BK,), jnp.int32),
        pltpu.VMEM((BK,), jnp.int32),
        pltpu.VMEM((BK,), jnp.float32),
        pltpu.VMEM((BK,), jnp.int32),
        pltpu.VMEM((BK,), jnp.int32),
        pltpu.VMEM((BK,), jnp.float32),
        pltpu.VMEM_SHARED((NP,), jnp.float32),
        pltpu.SemaphoreType.DMA,
    ],
)(_scatter_body)

_apply = functools.partial(
    pl.kernel,
    out_type=jax.ShapeDtypeStruct((NP,), jnp.float32),
    mesh=_MESH,
    compiler_params=_CPARAMS,
    scratch_types=[
        pltpu.VMEM((NPW,), jnp.float32),
        pltpu.VMEM((NPW,), jnp.int32),
        pltpu.VMEM((NPW,), jnp.float32),
        pltpu.VMEM((NPW,), jnp.float32),
        pltpu.VMEM((16,), jnp.int32),
        pltpu.SemaphoreType.DMA,
    ],
)(_apply_body)


def kernel(hdr, edge_index, node_level):
    src = edge_index[0]
    dst = edge_index[1]
    h = jnp.concatenate([hdr, jnp.zeros((NP - NN,), jnp.float32)])
    lvl = jnp.concatenate([node_level, jnp.zeros((NP - NN,), jnp.int32)])
    srcp = jnp.concatenate([src, jnp.zeros((EP - EE,), jnp.int32)])
    dstp = jnp.concatenate([dst, jnp.full((EP - EE,), NP - 1, jnp.int32)])
    neg = jnp.full((NP,), -jnp.inf, jnp.float32)
    for i in range(1, NLVL):
        aggs = _scatter(h, srcp, dstp, neg)
        h = _apply(h, aggs, lvl, jnp.full((16,), i, jnp.int32))
    return h[:NN]


# speedup vs baseline: 1.8370x; 1.8370x over previous
import functools

import jax
import jax.numpy as jnp
from jax import lax
from jax.experimental import pallas as pl
from jax.experimental.pallas import tpu as pltpu
from jax.experimental.pallas import tpu_sc as plsc

NN = 100000
NP = 102400
EE = 6400000
EP = 6553600
NW = 32
EPW = EP // NW
BK = 2048
NB = EPW // BK
CH = 128
NCH = BK // CH
NPW = NP // NW
NLVL = 8


def _take16(x, idx):
    return lax.gather(
        x, idx[:, None],
        dimension_numbers=lax.GatherDimensionNumbers(
            offset_dims=(), collapsed_slice_dims=(0,), start_index_map=(0,)),
        slice_sizes=(1,),
        mode=lax.GatherScatterMode.PROMISE_IN_BOUNDS)


NB0 = 100
NB1 = 100


def _scatter_body(h_hbm, src_hbm, dst_hbm, neg_hbm, out_hbm,
                  agg, srcb0, dstb0, msgb0, srcb1, dstb1, msgb1, hsh, sem):
    cid = lax.axis_index("c")
    sid = lax.axis_index("s")
    wid = sid * 2 + cid
    base_blk = sid * (NB0 + NB1) + cid * NB0
    nblk = NB0 + cid * (NB1 - NB0)
    base = base_blk * BK

    @pl.when(sid == 0)
    def _():
        pltpu.sync_copy(h_hbm, hsh)

    pltpu.sync_copy(neg_hbm, agg)
    plsc.subcore_barrier()
    iota = lax.iota(jnp.int32, 16)
    bufs = ((srcb0, dstb0, msgb0), (srcb1, dstb1, msgb1))

    def _stage(b, p):
        sb, db, mb = bufs[p]
        off = base + b * BK
        pltpu.sync_copy(src_hbm.at[pl.ds(off, BK)], sb)
        pltpu.sync_copy(dst_hbm.at[pl.ds(off, BK)], db)
        for c in range(NCH):
            pltpu.async_copy(hsh.at[sb.at[pl.ds(c * CH, CH)]],
                             mb.at[pl.ds(c * CH, CH)], sem)

    def _wait(p):
        sb, db, mb = bufs[p]
        for c in range(NCH):
            pltpu.make_async_copy(hsh.at[sb.at[pl.ds(c * CH, CH)]],
                                  mb.at[pl.ds(c * CH, CH)], sem).wait()

    def _compute(p):
        db, mb = bufs[p][1], bufs[p][2]

        def vec(j, _):
            d = db[pl.ds(j * 16, 16)]
            m = mb[pl.ds(j * 16, 16)] + 1.0
            ms, ds = plsc.sort_key_val(m, d)
            old = plsc.load_gather(agg, [ds])
            plsc.store_scatter(agg, [ds], jnp.maximum(old, ms))
            return 0

        lax.fori_loop(0, BK // 16, vec, 0)

    _stage(0, 0)

    def pair(t, _):
        for phase in range(2):
            b = t * 2 + phase
            _wait(phase)

            @pl.when(b + 1 < nblk)
            def _():
                _stage(b + 1, 1 - phase)

            _compute(phase)
        return 0

    lax.fori_loop(0, nblk // 2, pair, 0)
    pltpu.sync_copy(agg, out_hbm.at[wid])


def _apply_body(h_hbm, aggs_hbm, lvl_hbm, ivec_hbm, out_hbm,
                hbuf, lbuf, acc, tmp, ivec, sem):
    wid = lax.axis_index("s") * 2 + lax.axis_index("c")
    base = wid * NPW
    pltpu.sync_copy(h_hbm.at[pl.ds(base, NPW)], hbuf)
    pltpu.sync_copy(lvl_hbm.at[pl.ds(base, NPW)], lbuf)
    pltpu.sync_copy(ivec_hbm, ivec)
    pltpu.sync_copy(aggs_hbm.at[0, pl.ds(base, NPW)], acc)
    for s in range(1, NW):
        pltpu.sync_copy(aggs_hbm.at[s, pl.ds(base, NPW)], tmp)

        def mx(j, _):
            sl = pl.ds(j * 16, 16)
            acc[sl] = jnp.maximum(acc[sl], tmp[sl])
            return 0

        lax.fori_loop(0, NPW // 16, mx, 0)
    iv = ivec[...]

    def sel(j, _):
        sl = pl.ds(j * 16, 16)
        hbuf[sl] = jnp.where(lbuf[sl] == iv, acc[sl], hbuf[sl])
        return 0

    lax.fori_loop(0, NPW // 16, sel, 0)
    pltpu.sync_copy(hbuf, out_hbm.at[pl.ds(base, NPW)])


_MESH = plsc.VectorSubcoreMesh(core_axis_name="c", subcore_axis_name="s")
_CPARAMS = pltpu.CompilerParams(needs_layout_passes=False)

_scatter = functools.partial(
    pl.kernel,
    out_type=jax.ShapeDtypeStruct((NW, NP), jnp.float32),
    mesh=_MESH,
    compiler_params=_CPARAMS,
    scratch_types=[
        pltpu.VMEM((NP,), jnp.float32),
        pltpu.VMEM((BK,), jnp.int32),
        pltpu.VMEM((BK,), jnp.int32),
        pltpu.VMEM((BK,), jnp.float32),
        pltpu.VMEM((BK,), jnp.int32),
        pltpu.VMEM((BK,), jnp.int32),
        pltpu.VMEM((BK,), jnp.float32),
        pltpu.VMEM_SHARED((NP,), jnp.float32),
        pltpu.SemaphoreType.DMA,
    ],
)(_scatter_body)

_apply = functools.partial(
    pl.kernel,
    out_type=jax.ShapeDtypeStruct((NP,), jnp.float32),
    mesh=_MESH,
    compiler_params=_CPARAMS,
    scratch_types=[
        pltpu.VMEM((NPW,), jnp.float32),
        pltpu.VMEM((NPW,), jnp.int32),
        pltpu.VMEM((NPW,), jnp.float32),
        pltpu.VMEM((NPW,), jnp.float32),
        pltpu.VMEM((16,), jnp.int32),
        pltpu.SemaphoreType.DMA,
    ],
)(_apply_body)


def kernel(hdr, edge_index, node_level):
    src = edge_index[0]
    dst = edge_index[1]
    h = jnp.concatenate([hdr, jnp.zeros((NP - NN,), jnp.float32)])
    lvl = jnp.concatenate([node_level, jnp.zeros((NP - NN,), jnp.int32)])
    srcp = jnp.concatenate([src, jnp.zeros((EP - EE,), jnp.int32)])
    dstp = jnp.concatenate([dst, jnp.full((EP - EE,), NP - 1, jnp.int32)])
    neg = jnp.full((NP,), -jnp.inf, jnp.float32)
    for i in range(1, NLVL):
        aggs = _scatter(h, srcp, dstp, neg)
        h = _apply(h, aggs, lvl, jnp.full((16,), i, jnp.int32))
    return h[:NN]


# speedup vs baseline: 2.0149x; 1.0969x over previous
import functools

import jax
import jax.numpy as jnp
from jax import lax
from jax.experimental import pallas as pl
from jax.experimental.pallas import tpu as pltpu
from jax.experimental.pallas import tpu_sc as plsc

NN = 100000
NP = 102400
EE = 6400000
EP = 6553600
NW = 32
EPW = EP // NW
BK = 2048
NB = EPW // BK
CH = 128
NCH = BK // CH
NPW = NP // NW
NLVL = 8


def _take16(x, idx):
    return lax.gather(
        x, idx[:, None],
        dimension_numbers=lax.GatherDimensionNumbers(
            offset_dims=(), collapsed_slice_dims=(0,), start_index_map=(0,)),
        slice_sizes=(1,),
        mode=lax.GatherScatterMode.PROMISE_IN_BOUNDS)


NB0 = 100
NB1 = 100


def _scatter_body(h_hbm, src_hbm, dst_hbm, neg_hbm, out_hbm,
                  agg, srcb0, dstb0, msgb0, srcb1, dstb1, msgb1, hsh, sem):
    cid = lax.axis_index("c")
    sid = lax.axis_index("s")
    wid = sid * 2 + cid
    base_blk = sid * (NB0 + NB1) + cid * NB0
    nblk = NB0 + cid * (NB1 - NB0)
    base = base_blk * BK

    @pl.when(sid == 0)
    def _():
        pltpu.sync_copy(h_hbm, hsh)

    pltpu.sync_copy(neg_hbm, agg)
    plsc.subcore_barrier()
    iota = lax.iota(jnp.int32, 16)
    bufs = ((srcb0, dstb0, msgb0), (srcb1, dstb1, msgb1))

    def _stage(b, p):
        sb, db, mb = bufs[p]
        off = base + b * BK
        pltpu.sync_copy(src_hbm.at[pl.ds(off, BK)], sb)
        pltpu.sync_copy(dst_hbm.at[pl.ds(off, BK)], db)
        for c in range(NCH):
            pltpu.async_copy(hsh.at[sb.at[pl.ds(c * CH, CH)]],
                             mb.at[pl.ds(c * CH, CH)], sem)

    def _wait(p):
        sb, db, mb = bufs[p]
        for c in range(NCH):
            pltpu.make_async_copy(hsh.at[sb.at[pl.ds(c * CH, CH)]],
                                  mb.at[pl.ds(c * CH, CH)], sem).wait()

    def _compute(p):
        db, mb = bufs[p][1], bufs[p][2]

        def vec(j, _):
            d = db[pl.ds(j * 16, 16)]
            m = mb[pl.ds(j * 16, 16)] + 1.0
            ms, ds = plsc.sort_key_val(m, d)
            old = plsc.load_gather(agg, [ds])
            plsc.store_scatter(agg, [ds], jnp.maximum(old, ms))
            return 0

        lax.fori_loop(0, BK // 16, vec, 0)

    _stage(0, 0)

    def pair(t, _):
        for phase in range(2):
            b = t * 2 + phase
            _wait(phase)

            @pl.when(b + 1 < nblk)
            def _():
                _stage(b + 1, 1 - phase)

            _compute(phase)
        return 0

    lax.fori_loop(0, nblk // 2, pair, 0)
    pltpu.sync_copy(agg, out_hbm.at[wid])


def _apply_body(h_hbm, aggs_hbm, lvl_hbm, ivec_hbm, out_hbm,
                hbuf, lbuf, rows, ivec, sem):
    wid = lax.axis_index("s") * 2 + lax.axis_index("c")
    base = wid * NPW
    descs = [
        pltpu.async_copy(aggs_hbm.at[s, pl.ds(base, NPW)], rows.at[s], sem)
        for s in range(NW)
    ]
    pltpu.sync_copy(h_hbm.at[pl.ds(base, NPW)], hbuf)
    pltpu.sync_copy(lvl_hbm.at[pl.ds(base, NPW)], lbuf)
    pltpu.sync_copy(ivec_hbm, ivec)
    for dsc in descs:
        dsc.wait()
    iv = ivec[...]

    def sel(j, _):
        sl = pl.ds(j * 16, 16)
        acc = rows[0, sl]
        for s in range(1, NW):
            acc = jnp.maximum(acc, rows[s, sl])
        hbuf[sl] = jnp.where(lbuf[sl] == iv, acc, hbuf[sl])
        return 0

    lax.fori_loop(0, NPW // 16, sel, 0)
    pltpu.sync_copy(hbuf, out_hbm.at[pl.ds(base, NPW)])


_MESH = plsc.VectorSubcoreMesh(core_axis_name="c", subcore_axis_name="s")
_CPARAMS = pltpu.CompilerParams(needs_layout_passes=False)

_scatter = functools.partial(
    pl.kernel,
    out_type=jax.ShapeDtypeStruct((NW, NP), jnp.float32),
    mesh=_MESH,
    compiler_params=_CPARAMS,
    scratch_types=[
        pltpu.VMEM((NP,), jnp.float32),
        pltpu.VMEM((BK,), jnp.int32),
        pltpu.VMEM((BK,), jnp.int32),
        pltpu.VMEM((BK,), jnp.float32),
        pltpu.VMEM((BK,), jnp.int32),
        pltpu.VMEM((BK,), jnp.int32),
        pltpu.VMEM((BK,), jnp.float32),
        pltpu.VMEM_SHARED((NP,), jnp.float32),
        pltpu.SemaphoreType.DMA,
    ],
)(_scatter_body)

_apply = functools.partial(
    pl.kernel,
    out_type=jax.ShapeDtypeStruct((NP,), jnp.float32),
    mesh=_MESH,
    compiler_params=_CPARAMS,
    scratch_types=[
        pltpu.VMEM((NPW,), jnp.float32),
        pltpu.VMEM((NPW,), jnp.int32),
        pltpu.VMEM((NW, NPW), jnp.float32),
        pltpu.VMEM((16,), jnp.int32),
        pltpu.SemaphoreType.DMA,
    ],
)(_apply_body)


def kernel(hdr, edge_index, node_level):
    src = edge_index[0]
    dst = edge_index[1]
    h = jnp.concatenate([hdr, jnp.zeros((NP - NN,), jnp.float32)])
    lvl = jnp.concatenate([node_level, jnp.zeros((NP - NN,), jnp.int32)])
    srcp = jnp.concatenate([src, jnp.zeros((EP - EE,), jnp.int32)])
    dstp = jnp.concatenate([dst, jnp.full((EP - EE,), NP - 1, jnp.int32)])
    neg = jnp.full((NP,), -jnp.inf, jnp.float32)
    for i in range(1, NLVL):
        aggs = _scatter(h, srcp, dstp, neg)
        h = _apply(h, aggs, lvl, jnp.full((16,), i, jnp.int32))
    return h[:NN]


# speedup vs baseline: 4.4912x; 2.2290x over previous
import functools

import jax
import jax.numpy as jnp
from jax import lax
from jax.experimental import pallas as pl
from jax.experimental.pallas import tpu as pltpu
from jax.experimental.pallas import tpu_sc as plsc

NN = 100000
NP = 102400
EE = 6400000
EP = 6553600
NW = 32
EPW = EP // NW
BK = 2048
NB = EPW // BK
CH = 128
NCH = BK // CH
NPW = NP // NW
NLVL = 8
SW = BK + 16
RST = EPW + 2 * BK


def _take16(x, idx):
    return lax.gather(
        x, idx[:, None],
        dimension_numbers=lax.GatherDimensionNumbers(
            offset_dims=(), collapsed_slice_dims=(0,), start_index_map=(0,)),
        slice_sizes=(1,),
        mode=lax.GatherScatterMode.PROMISE_IN_BOUNDS)


def _bucket_body(src_hbm, dst_hbm, lvl_hbm, bsrc_hbm, bdst_hbm, cnts_hbm,
                 srcb, dstb, lvlb, stgs, stgd, fillm, cvec, lvlsh,
                 semg, semf):
    cid = lax.axis_index("c")
    sid = lax.axis_index("s")
    wid = sid * 2 + cid
    base = wid * EPW

    @pl.when(sid == 0)
    def _():
        pltpu.sync_copy(lvl_hbm, lvlsh)

    iota = lax.iota(jnp.int32, 16)
    zv = jnp.zeros((16,), jnp.int32)
    dumv = jnp.full((16,), NP - 1, jnp.int32)

    def ms(j, _):
        stgs[pl.ds(NLVL * SW + j * 16, 16)] = zv
        stgd[pl.ds(NLVL * SW + j * 16, 16)] = dumv
        return 0

    lax.fori_loop(0, SW // 16, ms, 0)
    fillm[...] = zv
    plsc.subcore_barrier()

    def _flush(l, off):
        hb = pl.multiple_of((wid * NLVL + l) * RST + off, 16)
        pltpu.async_copy(stgs.at[pl.ds(l * SW, SW)],
                         bsrc_hbm.at[pl.ds(hb, SW)], semf)
        pltpu.async_copy(stgd.at[pl.ds(l * SW, SW)],
                         bdst_hbm.at[pl.ds(hb, SW)], semf)

    def _flush_wait(l, off):
        hb = pl.multiple_of((wid * NLVL + l) * RST + off, 16)
        pltpu.make_async_copy(stgs.at[pl.ds(l * SW, SW)],
                              bsrc_hbm.at[pl.ds(hb, SW)], semf).wait()
        pltpu.make_async_copy(stgd.at[pl.ds(l * SW, SW)],
                              bdst_hbm.at[pl.ds(hb, SW)], semf).wait()

    def block(b, carry):
        offs, prev = carry[:NLVL], carry[NLVL:]
        off_e = base + b * BK
        pltpu.sync_copy(src_hbm.at[pl.ds(off_e, BK)], srcb)
        pltpu.sync_copy(dst_hbm.at[pl.ds(off_e, BK)], dstb)
        descs = [
            pltpu.async_copy(lvlsh.at[dstb.at[pl.ds(c * CH, CH)]],
                             lvlb.at[pl.ds(c * CH, CH)], semg)
            for c in range(NCH)
        ]

        @pl.when(b > 0)
        def _():
            for l in range(NLVL):
                _flush_wait(l, prev[l])

        for dsc in descs:
            dsc.wait()

        def vec(j, _):
            sl = pl.ds(j * 16, 16)
            lv = lvlb[sl]
            sv = srcb[sl]
            dv = dstb[sl]
            ks, ids = plsc.sort_key_val(lv, iota)
            st = iota
            for s in (1, 2, 4, 8):
                idxs = jnp.maximum(iota - s, 0)
                kss = _take16(ks, idxs)
                sts = _take16(st, idxs)
                st = jnp.where((iota >= s) & (kss == ks),
                               jnp.minimum(st, sts), st)
            rank = iota - st
            kl = _take16(ks, jnp.minimum(iota + 1, 15))
            is_last = (ks != kl) | (iota == 15)
            fv = fillm[...]
            pos = _take16(fv, ks) + ks * SW + rank
            plsc.store_scatter(stgs, [pos], _take16(sv, ids))
            plsc.store_scatter(stgd, [pos], _take16(dv, ids))
            plsc.addupdate_scatter(fillm, [ks], rank + 1, mask=is_last)
            return 0

        lax.fori_loop(0, BK // 16, vec, 0)
        fv = fillm[...]
        newoffs = []
        for l in range(NLVL):
            fl = jnp.sum(jnp.where(iota == l, fv, 0))
            pad_idx = l * SW + fl + iota
            plsc.store_scatter(stgs, [pad_idx], zv)
            plsc.store_scatter(stgd, [pad_idx], dumv)
            _flush(l, offs[l])
            newoffs.append(offs[l] + (fl + 15) // 16 * 16)
        fillm[...] = zv
        return tuple(newoffs) + tuple(offs)

    carry = lax.fori_loop(0, NB, block, (jnp.int32(0),) * (2 * NLVL))
    offs, prev = carry[:NLVL], carry[NLVL:]
    for l in range(NLVL):
        _flush_wait(l, prev[l])
    cv = zv
    for l in range(NLVL):
        hb = pl.multiple_of((wid * NLVL + l) * RST + offs[l], 16)
        pltpu.sync_copy(stgs.at[pl.ds(NLVL * SW, SW)],
                        bsrc_hbm.at[pl.ds(hb, SW)])
        pltpu.sync_copy(stgd.at[pl.ds(NLVL * SW, SW)],
                        bdst_hbm.at[pl.ds(hb, SW)])
        cv = jnp.where(iota == l, offs[l], cv)
    cvec[...] = cv
    pltpu.sync_copy(cvec, cnts_hbm.at[wid])


def _scatter_body(lvl_i, h_hbm, bsrc_hbm, bdst_hbm, cnts_hbm, neg_hbm,
                  out_hbm, agg, srcb0, dstb0, msgb0, srcb1, dstb1, msgb1,
                  cvec, hsh, sem):
    cid = lax.axis_index("c")
    sid = lax.axis_index("s")
    wid = sid * 2 + cid
    base = (wid * NLVL + lvl_i) * RST

    @pl.when(sid == 0)
    def _():
        pltpu.sync_copy(h_hbm, hsh)

    pltpu.sync_copy(neg_hbm, agg)
    pltpu.sync_copy(cnts_hbm.at[wid], cvec)
    plsc.subcore_barrier()
    iota = lax.iota(jnp.int32, 16)
    cnt = jnp.sum(jnp.where(iota == lvl_i, cvec[...], 0))
    nblk = (cnt + BK - 1) // BK
    bufs = ((srcb0, dstb0, msgb0), (srcb1, dstb1, msgb1))

    def _stage(b, p):
        sb, db, mb = bufs[p]
        off = pl.multiple_of(base + b * BK, 16)
        pltpu.sync_copy(bsrc_hbm.at[pl.ds(off, BK)], sb)
        pltpu.sync_copy(bdst_hbm.at[pl.ds(off, BK)], db)
        for c in range(NCH):
            pltpu.async_copy(hsh.at[sb.at[pl.ds(c * CH, CH)]],
                             mb.at[pl.ds(c * CH, CH)], sem)

    def _wait(p):
        sb, db, mb = bufs[p]
        for c in range(NCH):
            pltpu.make_async_copy(hsh.at[sb.at[pl.ds(c * CH, CH)]],
                                  mb.at[pl.ds(c * CH, CH)], sem).wait()

    def _compute(p):
        db, mb = bufs[p][1], bufs[p][2]

        def vec(j, _):
            d = db[pl.ds(j * 16, 16)]
            m = mb[pl.ds(j * 16, 16)] + 1.0
            ms, ds = plsc.sort_key_val(m, d)
            old = plsc.load_gather(agg, [ds])
            plsc.store_scatter(agg, [ds], jnp.maximum(old, ms))
            return 0

        lax.fori_loop(0, BK // 16, vec, 0)

    @pl.when(nblk > 0)
    def _():
        _stage(0, 0)

    def pair(t, _):
        for phase in range(2):
            b = t * 2 + phase

            @pl.when(b < nblk)
            def _():
                _wait(phase)

                @pl.when(b + 1 < nblk)
                def _():
                    _stage(b + 1, 1 - phase)

                _compute(phase)

        return 0

    lax.fori_loop(0, (nblk + 1) // 2, pair, 0)
    pltpu.sync_copy(agg, out_hbm.at[wid])


def _apply_body(h_hbm, aggs_hbm, lvl_hbm, ivec_hbm, out_hbm,
                hbuf, lbuf, rows, ivec, sem):
    wid = lax.axis_index("s") * 2 + lax.axis_index("c")
    base = wid * NPW
    descs = [
        pltpu.async_copy(aggs_hbm.at[s, pl.ds(base, NPW)], rows.at[s], sem)
        for s in range(NW)
    ]
    pltpu.sync_copy(h_hbm.at[pl.ds(base, NPW)], hbuf)
    pltpu.sync_copy(lvl_hbm.at[pl.ds(base, NPW)], lbuf)
    pltpu.sync_copy(ivec_hbm, ivec)
    for dsc in descs:
        dsc.wait()
    iv = ivec[...]

    def sel(j, _):
        sl = pl.ds(j * 16, 16)
        acc = rows[0, sl]
        for s in range(1, NW):
            acc = jnp.maximum(acc, rows[s, sl])
        hbuf[sl] = jnp.where(lbuf[sl] == iv, acc, hbuf[sl])
        return 0

    lax.fori_loop(0, NPW // 16, sel, 0)
    pltpu.sync_copy(hbuf, out_hbm.at[pl.ds(base, NPW)])


_MESH = plsc.VectorSubcoreMesh(core_axis_name="c", subcore_axis_name="s")
_CPARAMS = pltpu.CompilerParams(needs_layout_passes=False)

_bucket = functools.partial(
    pl.kernel,
    out_type=(
        jax.ShapeDtypeStruct((NW * NLVL * RST,), jnp.int32),
        jax.ShapeDtypeStruct((NW * NLVL * RST,), jnp.int32),
        jax.ShapeDtypeStruct((NW, 16), jnp.int32),
    ),
    mesh=_MESH,
    compiler_params=_CPARAMS,
    scratch_types=[
        pltpu.VMEM((BK,), jnp.int32),
        pltpu.VMEM((BK,), jnp.int32),
        pltpu.VMEM((BK,), jnp.int32),
        pltpu.VMEM(((NLVL + 1) * SW,), jnp.int32),
        pltpu.VMEM(((NLVL + 1) * SW,), jnp.int32),
        pltpu.VMEM((16,), jnp.int32),
        pltpu.VMEM((16,), jnp.int32),
        pltpu.VMEM_SHARED((NP,), jnp.int32),
        pltpu.SemaphoreType.DMA,
        pltpu.SemaphoreType.DMA,
    ],
)(_bucket_body)

_SC_SCRATCH = [
    pltpu.VMEM((NP,), jnp.float32),
    pltpu.VMEM((BK,), jnp.int32),
    pltpu.VMEM((BK,), jnp.int32),
    pltpu.VMEM((BK,), jnp.float32),
    pltpu.VMEM((BK,), jnp.int32),
    pltpu.VMEM((BK,), jnp.int32),
    pltpu.VMEM((BK,), jnp.float32),
    pltpu.VMEM((16,), jnp.int32),
    pltpu.VMEM_SHARED((NP,), jnp.float32),
    pltpu.SemaphoreType.DMA,
]

_scatters = {
    i: functools.partial(
        pl.kernel,
        out_type=jax.ShapeDtypeStruct((NW, NP), jnp.float32),
        mesh=_MESH,
        compiler_params=_CPARAMS,
        scratch_types=_SC_SCRATCH,
    )(functools.partial(_scatter_body, i))
    for i in range(1, NLVL)
}

_apply = functools.partial(
    pl.kernel,
    out_type=jax.ShapeDtypeStruct((NP,), jnp.float32),
    mesh=_MESH,
    compiler_params=_CPARAMS,
    scratch_types=[
        pltpu.VMEM((NPW,), jnp.float32),
        pltpu.VMEM((NPW,), jnp.int32),
        pltpu.VMEM((NW, NPW), jnp.float32),
        pltpu.VMEM((16,), jnp.int32),
        pltpu.SemaphoreType.DMA,
    ],
)(_apply_body)


def kernel(hdr, edge_index, node_level):
    src = edge_index[0]
    dst = edge_index[1]
    h = jnp.concatenate([hdr, jnp.zeros((NP - NN,), jnp.float32)])
    lvl = jnp.concatenate([node_level, jnp.zeros((NP - NN,), jnp.int32)])
    srcp = jnp.concatenate([src, jnp.zeros((EP - EE,), jnp.int32)])
    dstp = jnp.concatenate([dst, jnp.full((EP - EE,), NP - 1, jnp.int32)])
    neg = jnp.full((NP,), -jnp.inf, jnp.float32)
    bsrc, bdst, cnts = _bucket(srcp, dstp, lvl)
    for i in range(1, NLVL):
        aggs = _scatters[i](h, bsrc, bdst, cnts, neg)
        h = _apply(h, aggs, lvl, jnp.full((16,), i, jnp.int32))
    return h[:NN]


# speedup vs baseline: 4.4938x; 1.0006x over previous
import functools

import jax
import jax.numpy as jnp
from jax import lax
from jax.experimental import pallas as pl
from jax.experimental.pallas import tpu as pltpu
from jax.experimental.pallas import tpu_sc as plsc

NN = 100000
NP = 102400
EE = 6400000
EP = 6553600
NW = 32
EPW = EP // NW
BK = 2048
NB = EPW // BK
CH = 128
NCH = BK // CH
NPW = NP // NW
NLVL = 8
SW = BK + 16
RST = EPW + 2 * BK


def _take16(x, idx):
    return lax.gather(
        x, idx[:, None],
        dimension_numbers=lax.GatherDimensionNumbers(
            offset_dims=(), collapsed_slice_dims=(0,), start_index_map=(0,)),
        slice_sizes=(1,),
        mode=lax.GatherScatterMode.PROMISE_IN_BOUNDS)


def _bucket_body(src_hbm, dst_hbm, lvl_hbm, bsrc_hbm, bdst_hbm, cnts_hbm,
                 srcb, dstb, lvlb, stgs, stgd, fillm, cvec, lvlsh,
                 semg, semf):
    cid = lax.axis_index("c")
    sid = lax.axis_index("s")
    wid = sid * 2 + cid
    base = wid * EPW

    @pl.when(sid == 0)
    def _():
        pltpu.sync_copy(lvl_hbm, lvlsh)

    iota = lax.iota(jnp.int32, 16)
    zv = jnp.zeros((16,), jnp.int32)
    dumv = jnp.full((16,), NP - 1, jnp.int32)

    def ms(j, _):
        stgs[pl.ds(NLVL * SW + j * 16, 16)] = zv
        stgd[pl.ds(NLVL * SW + j * 16, 16)] = dumv
        return 0

    lax.fori_loop(0, SW // 16, ms, 0)
    fillm[...] = zv
    plsc.subcore_barrier()

    def _flush(l, off):
        hb = pl.multiple_of((wid * NLVL + l) * RST + off, 16)
        pltpu.async_copy(stgs.at[pl.ds(l * SW, SW)],
                         bsrc_hbm.at[pl.ds(hb, SW)], semf)
        pltpu.async_copy(stgd.at[pl.ds(l * SW, SW)],
                         bdst_hbm.at[pl.ds(hb, SW)], semf)

    def _flush_wait(l, off):
        hb = pl.multiple_of((wid * NLVL + l) * RST + off, 16)
        pltpu.make_async_copy(stgs.at[pl.ds(l * SW, SW)],
                              bsrc_hbm.at[pl.ds(hb, SW)], semf).wait()
        pltpu.make_async_copy(stgd.at[pl.ds(l * SW, SW)],
                              bdst_hbm.at[pl.ds(hb, SW)], semf).wait()

    def block(b, carry):
        offs, prev = carry[:NLVL], carry[NLVL:]
        off_e = base + b * BK
        pltpu.sync_copy(src_hbm.at[pl.ds(off_e, BK)], srcb)
        pltpu.sync_copy(dst_hbm.at[pl.ds(off_e, BK)], dstb)
        descs = [
            pltpu.async_copy(lvlsh.at[dstb.at[pl.ds(c * CH, CH)]],
                             lvlb.at[pl.ds(c * CH, CH)], semg)
            for c in range(NCH)
        ]

        @pl.when(b > 0)
        def _():
            for l in range(NLVL):
                _flush_wait(l, prev[l])

        for dsc in descs:
            dsc.wait()

        def vec(j, _):
            sl = pl.ds(j * 16, 16)
            lv = lvlb[sl]
            sv = srcb[sl]
            dv = dstb[sl]
            ks, ids = plsc.sort_key_val(lv, iota)
            st = iota
            for s in (1, 2, 4, 8):
                idxs = jnp.maximum(iota - s, 0)
                kss = _take16(ks, idxs)
                sts = _take16(st, idxs)
                st = jnp.where((iota >= s) & (kss == ks),
                               jnp.minimum(st, sts), st)
            rank = iota - st
            kl = _take16(ks, jnp.minimum(iota + 1, 15))
            is_last = (ks != kl) | (iota == 15)
            fv = fillm[...]
            pos = _take16(fv, ks) + ks * SW + rank
            plsc.store_scatter(stgs, [pos], _take16(sv, ids))
            plsc.store_scatter(stgd, [pos], _take16(dv, ids))
            plsc.addupdate_scatter(fillm, [ks], rank + 1, mask=is_last)
            return 0

        lax.fori_loop(0, BK // 16, vec, 0)
        fv = fillm[...]
        newoffs = []
        for l in range(NLVL):
            fl = jnp.sum(jnp.where(iota == l, fv, 0))
            pad_idx = l * SW + fl + iota
            plsc.store_scatter(stgs, [pad_idx], zv)
            plsc.store_scatter(stgd, [pad_idx], dumv)
            _flush(l, offs[l])
            newoffs.append(offs[l] + (fl + 15) // 16 * 16)
        fillm[...] = zv
        return tuple(newoffs) + tuple(offs)

    carry = lax.fori_loop(0, NB, block, (jnp.int32(0),) * (2 * NLVL))
    offs, prev = carry[:NLVL], carry[NLVL:]
    for l in range(NLVL):
        _flush_wait(l, prev[l])
    cv = zv
    for l in range(NLVL):
        hb = pl.multiple_of((wid * NLVL + l) * RST + offs[l], 16)
        pltpu.sync_copy(stgs.at[pl.ds(NLVL * SW, SW)],
                        bsrc_hbm.at[pl.ds(hb, SW)])
        pltpu.sync_copy(stgd.at[pl.ds(NLVL * SW, SW)],
                        bdst_hbm.at[pl.ds(hb, SW)])
        cv = jnp.where(iota == l, offs[l], cv)
    cvec[...] = cv
    pltpu.sync_copy(cvec, cnts_hbm.at[wid])


def _scatter_body(lvl_i, h_hbm, bsrc_hbm, bdst_hbm, cnts_hbm, neg_hbm,
                  out_hbm, agg, srcb0, dstb0, msgb0, srcb1, dstb1, msgb1,
                  cvec, hsh, sem):
    cid = lax.axis_index("c")
    sid = lax.axis_index("s")
    wid = sid * 2 + cid
    base = (wid * NLVL + lvl_i) * RST

    @pl.when(sid == 0)
    def _():
        pltpu.sync_copy(h_hbm, hsh)

    pltpu.sync_copy(neg_hbm, agg)
    pltpu.sync_copy(cnts_hbm.at[wid], cvec)
    plsc.subcore_barrier()
    iota = lax.iota(jnp.int32, 16)
    cnt = jnp.sum(jnp.where(iota == lvl_i, cvec[...], 0))
    nblk = (cnt + BK - 1) // BK
    bufs = ((srcb0, dstb0, msgb0), (srcb1, dstb1, msgb1))

    def _stage(b, p):
        sb, db, mb = bufs[p]
        off = pl.multiple_of(base + b * BK, 16)
        pltpu.sync_copy(bsrc_hbm.at[pl.ds(off, BK)], sb)
        pltpu.sync_copy(bdst_hbm.at[pl.ds(off, BK)], db)
        for c in range(NCH):
            pltpu.async_copy(hsh.at[sb.at[pl.ds(c * CH, CH)]],
                             mb.at[pl.ds(c * CH, CH)], sem)

    def _wait(p):
        sb, db, mb = bufs[p]
        for c in range(NCH):
            pltpu.make_async_copy(hsh.at[sb.at[pl.ds(c * CH, CH)]],
                                  mb.at[pl.ds(c * CH, CH)], sem).wait()

    def _compute(p):
        db, mb = bufs[p][1], bufs[p][2]

        def vec(j, _):
            d = db[pl.ds(j * 16, 16)]
            m = mb[pl.ds(j * 16, 16)] + 1.0
            ms, ds = plsc.sort_key_val(m, d)
            old = plsc.load_gather(agg, [ds])
            plsc.store_scatter(agg, [ds], jnp.maximum(old, ms))
            return 0

        lax.fori_loop(0, BK // 16, vec, 0)

    @pl.when(nblk > 0)
    def _():
        _stage(0, 0)

    def pair(t, _):
        for phase in range(2):
            b = t * 2 + phase

            @pl.when(b < nblk)
            def _():
                _wait(phase)

                @pl.when(b + 1 < nblk)
                def _():
                    _stage(b + 1, 1 - phase)

                _compute(phase)

        return 0

    lax.fori_loop(0, (nblk + 1) // 2, pair, 0)
    pltpu.sync_copy(agg, out_hbm.at[wid])


def _apply_body(h_hbm, aggs_hbm, lvl_hbm, ivec_hbm, out_hbm,
                hbuf, lbuf, rows, ivec, sem):
    wid = lax.axis_index("s") * 2 + lax.axis_index("c")
    base = wid * NPW
    descs = [
        pltpu.async_copy(aggs_hbm.at[s, pl.ds(base, NPW)], rows.at[s], sem)
        for s in range(NW)
    ]
    pltpu.sync_copy(h_hbm.at[pl.ds(base, NPW)], hbuf)
    pltpu.sync_copy(lvl_hbm.at[pl.ds(base, NPW)], lbuf)
    pltpu.sync_copy(ivec_hbm, ivec)
    for dsc in descs:
        dsc.wait()
    iv = ivec[...]

    def sel(j, _):
        sl = pl.ds(j * 16, 16)
        acc = rows[0, sl]
        for s in range(1, NW):
            acc = jnp.maximum(acc, rows[s, sl])
        hbuf[sl] = jnp.where(lbuf[sl] == iv, acc, hbuf[sl])
        return 0

    lax.fori_loop(0, NPW // 16, sel, 0)
    pltpu.sync_copy(hbuf, out_hbm.at[pl.ds(base, NPW)])


_MESH = plsc.VectorSubcoreMesh(core_axis_name="c", subcore_axis_name="s",
                               num_cores=2, num_subcores=16)
_CPARAMS = pltpu.CompilerParams(needs_layout_passes=False)

_bucket = functools.partial(
    pl.kernel,
    out_type=(
        jax.ShapeDtypeStruct((NW * NLVL * RST,), jnp.int32),
        jax.ShapeDtypeStruct((NW * NLVL * RST,), jnp.int32),
        jax.ShapeDtypeStruct((NW, 16), jnp.int32),
    ),
    mesh=_MESH,
    compiler_params=_CPARAMS,
    scratch_types=[
        pltpu.VMEM((BK,), jnp.int32),
        pltpu.VMEM((BK,), jnp.int32),
        pltpu.VMEM((BK,), jnp.int32),
        pltpu.VMEM(((NLVL + 1) * SW,), jnp.int32),
        pltpu.VMEM(((NLVL + 1) * SW,), jnp.int32),
        pltpu.VMEM((16,), jnp.int32),
        pltpu.VMEM((16,), jnp.int32),
        pltpu.VMEM_SHARED((NP,), jnp.int32),
        pltpu.SemaphoreType.DMA,
        pltpu.SemaphoreType.DMA,
    ],
)(_bucket_body)

_SC_SCRATCH = [
    pltpu.VMEM((NP,), jnp.float32),
    pltpu.VMEM((BK,), jnp.int32),
    pltpu.VMEM((BK,), jnp.int32),
    pltpu.VMEM((BK,), jnp.float32),
    pltpu.VMEM((BK,), jnp.int32),
    pltpu.VMEM((BK,), jnp.int32),
    pltpu.VMEM((BK,), jnp.float32),
    pltpu.VMEM((16,), jnp.int32),
    pltpu.VMEM_SHARED((NP,), jnp.float32),
    pltpu.SemaphoreType.DMA,
]

_scatters = {
    i: functools.partial(
        pl.kernel,
        out_type=jax.ShapeDtypeStruct((NW, NP), jnp.float32),
        mesh=_MESH,
        compiler_params=_CPARAMS,
        scratch_types=_SC_SCRATCH,
    )(functools.partial(_scatter_body, i))
    for i in range(1, NLVL)
}

_apply = functools.partial(
    pl.kernel,
    out_type=jax.ShapeDtypeStruct((NP,), jnp.float32),
    mesh=_MESH,
    compiler_params=_CPARAMS,
    scratch_types=[
        pltpu.VMEM((NPW,), jnp.float32),
        pltpu.VMEM((NPW,), jnp.int32),
        pltpu.VMEM((NW, NPW), jnp.float32),
        pltpu.VMEM((16,), jnp.int32),
        pltpu.SemaphoreType.DMA,
    ],
)(_apply_body)


def kernel(hdr, edge_index, node_level):
    src = edge_index[0]
    dst = edge_index[1]
    h = jnp.concatenate([hdr, jnp.zeros((NP - NN,), jnp.float32)])
    lvl = jnp.concatenate([node_level, jnp.zeros((NP - NN,), jnp.int32)])
    srcp = jnp.concatenate([src, jnp.zeros((EP - EE,), jnp.int32)])
    dstp = jnp.concatenate([dst, jnp.full((EP - EE,), NP - 1, jnp.int32)])
    neg = jnp.full((NP,), -jnp.inf, jnp.float32)
    bsrc, bdst, cnts = _bucket(srcp, dstp, lvl)
    for i in range(1, NLVL):
        aggs = _scatters[i](h, bsrc, bdst, cnts, neg)
        h = _apply(h, aggs, lvl, jnp.full((16,), i, jnp.int32))
    return h[:NN]
